# baseline (speedup 1.0000x reference)
import functools

import jax
import jax.numpy as jnp
from jax import lax
from jax.experimental import pallas as pl
from jax.experimental.pallas import tpu as pltpu
from jax.experimental.pallas import tpu_sc as plsc

D = 64
V0, V1, V2 = 22, 6, 2
VC = V0 * V1 * V2
NC, NS, L = 2, 16, 16
NW = NC * NS
NQ = NW // 8
BKJ = 25
NSLOT = 3


@functools.lru_cache(maxsize=None)
def _make_gather(E):
    NJ = E // 128
    assert NJ * 128 == E and NJ % (NQ * BKJ) == 0
    nj_q = NJ // NQ
    nb = nj_q // BKJ
    mesh = plsc.VectorSubcoreMesh(core_axis_name="c", subcore_axis_name="s")

    @functools.partial(
        pl.kernel,
        out_type=jax.ShapeDtypeStruct((8, NJ, 8, 128), jnp.float32),
        mesh=mesh,
        scratch_types=[
            pltpu.VMEM((V0 * D,), jnp.float32),
            pltpu.VMEM((V1 * D,), jnp.float32),
            pltpu.VMEM((V2 * D,), jnp.float32),
            pltpu.VMEM((D * VC,), jnp.float32),
            pltpu.VMEM((NSLOT, BKJ * 128), jnp.int32),
            pltpu.VMEM((NSLOT, BKJ, 8, 128), jnp.float32),
            pltpu.SemaphoreType.DMA,
            pltpu.SemaphoreType.DMA,
            pltpu.SemaphoreType.DMA,
            pltpu.SemaphoreType.DMA,
            pltpu.SemaphoreType.DMA,
            pltpu.SemaphoreType.DMA,
        ],
        compiler_params=pltpu.CompilerParams(
            needs_layout_passes=False, use_tc_tiling_on_sc=False
        ),
    )
    def _gather(e0_hbm, e1_hbm, e2_hbm, f_hbm, out_hbm,
                e0v, e1v, e2v, tbl, av, ob,
                si0, si1, si2, so0, so1, so2):
        wid = lax.axis_index("s") * NC + lax.axis_index("c")
        irow = wid // NQ
        jq = wid % NQ
        j0 = jq * nj_q
        doff = irow * (8 * VC)
        sis = [si0, si1, si2]
        sos = [so0, so1, so2]
        iota = lax.iota(jnp.int32, L)
        iota_vc = iota * VC

        pltpu.sync_copy(e0_hbm, e0v)
        pltpu.sync_copy(e1_hbm, e1v)
        pltpu.sync_copy(e2_hbm, e2v)

        def bi(i, carry):
            def bj(j, carry):
                def bk(k, carry):
                    v = i * (V1 * V2) + j * V2 + k
                    for c in range(D // L):
                        row = (e0v[pl.ds(i * D + c * L, L)]
                               + e1v[pl.ds(j * D + c * L, L)]
                               + e2v[pl.ds(k * D + c * L, L)])
                        plsc.store_scatter(tbl, [iota_vc + (c * L * VC + v)], row)
                    return carry
                return lax.fori_loop(0, V2, bk, 0)
            return lax.fori_loop(0, V1, bj, 0)
        lax.fori_loop(0, V0, bi, 0)

        def issue_in(b, s):
            pltpu.async_copy(
                f_hbm.at[pl.ds((j0 + b * BKJ) * 128, BKJ * 128)],
                av.at[s], sis[s],
            )

        def wait_in(s):
            pltpu.make_async_copy(
                f_hbm.at[pl.ds(0, BKJ * 128)], av.at[s], sis[s]
            ).wait()

        def issue_out(b, s):
            pltpu.async_copy(
                ob.at[s], out_hbm.at[irow, pl.ds(j0 + b * BKJ, BKJ)], sos[s]
            )

        def wait_out(s):
            pltpu.make_async_copy(
                ob.at[s], out_hbm.at[0, pl.ds(0, BKJ)], sos[s]
            ).wait()

        for s in range(NSLOT):
            issue_in(s, s)

        def body(h, carry):
            for s in range(NSLOT):
                b = h * NSLOT + s

                @pl.when(b < nb)
                def _(b=b, s=s):
                    wait_in(s)

                    @pl.when(b >= NSLOT)
                    def _():
                        wait_out(s)

                    def grp(g, carry, s=s):
                        jj = g >> 3
                        c16 = (g & 7) * L
                        base = av[s, pl.ds(g * L, L)] + doff
                        ws = [plsc.load_gather(tbl, [base + r * VC])
                              for r in range(8)]
                        for r in range(8):
                            ob[s, jj, r, pl.ds(c16, L)] = ws[r]
                        return carry
                    lax.fori_loop(0, BKJ * 8, grp, 0)

                    issue_out(b, s)

                    @pl.when(b + NSLOT < nb)
                    def _():
                        issue_in(b + NSLOT, s)
            return carry

        lax.fori_loop(0, (nb + NSLOT - 1) // NSLOT, body, 0)

        for s in range(NSLOT):
            @pl.when(nb > s)
            def _(s=s):
                wait_out(s)

    return _gather


@jax.jit
def kernel(edge_attr, emb0, emb1, emb2):
    E = edge_attr.shape[0]
    ea = edge_attr.astype(jnp.int32)
    f = ea[:, 0] * (V1 * V2) + ea[:, 1] * V2 + ea[:, 2]
    m = _make_gather(E)(
        emb0.reshape(V0 * D), emb1.reshape(V1 * D), emb2.reshape(V2 * D), f,
    )
    return m.transpose(1, 3, 0, 2).reshape(E, D)


# speedup vs baseline: 1.4181x; 1.4181x over previous
import functools

import jax
import jax.numpy as jnp
from jax import lax
from jax.experimental import pallas as pl
from jax.experimental.pallas import tpu as pltpu
from jax.experimental.pallas import tpu_sc as plsc

D = 64
V0, V1, V2 = 22, 6, 2
VC = V0 * V1 * V2
NC, NS, L = 2, 16, 16
NW = NC * NS
BKJ = 4
NSLOT = 3


@functools.lru_cache(maxsize=None)
def _make_gather(E):
    NJ = E // 128
    assert NJ * 128 == E and NJ % BKJ == 0
    NB = NJ // BKJ
    base_nb = NB // NW
    extra = NB - base_nb * NW
    mesh = plsc.VectorSubcoreMesh(core_axis_name="c", subcore_axis_name="s")

    @functools.partial(
        pl.kernel,
        out_type=jax.ShapeDtypeStruct((8, NJ, 8, 128), jnp.float32),
        mesh=mesh,
        scratch_types=[
            pltpu.VMEM((V0 * D,), jnp.float32),
            pltpu.VMEM((V1 * D,), jnp.float32),
            pltpu.VMEM((V2 * D,), jnp.float32),
            pltpu.VMEM((D * VC,), jnp.float32),
            pltpu.VMEM((NSLOT, BKJ * 128), jnp.int32),
            pltpu.VMEM((NSLOT, 8, BKJ, 8, 128), jnp.float32),
            pltpu.SemaphoreType.DMA,
            pltpu.SemaphoreType.DMA,
            pltpu.SemaphoreType.DMA,
            pltpu.SemaphoreType.DMA,
            pltpu.SemaphoreType.DMA,
            pltpu.SemaphoreType.DMA,
        ],
        compiler_params=pltpu.CompilerParams(
            needs_layout_passes=False, use_tc_tiling_on_sc=False
        ),
    )
    def _gather(e0_hbm, e1_hbm, e2_hbm, f_hbm, out_hbm,
                e0v, e1v, e2v, tbl, av, ob,
                si0, si1, si2, so0, so1, so2):
        wid = lax.axis_index("s") * NC + lax.axis_index("c")
        lo = wid * base_nb + lax.min(wid, extra)
        nb = base_nb + jnp.where(wid < extra, 1, 0).astype(jnp.int32)
        sis = [si0, si1, si2]
        sos = [so0, so1, so2]
        iota = lax.iota(jnp.int32, L)
        iota_vc = iota * VC

        def issue_in(b, s):
            pltpu.async_copy(
                f_hbm.at[pl.ds((lo + b) * (BKJ * 128), BKJ * 128)],
                av.at[s], sis[s],
            )

        def wait_in(s):
            pltpu.make_async_copy(
                f_hbm.at[pl.ds(0, BKJ * 128)], av.at[s], sis[s]
            ).wait()

        def issue_out(b, s):
            pltpu.async_copy(
                ob.at[s], out_hbm.at[:, pl.ds((lo + b) * BKJ, BKJ)], sos[s]
            )

        def wait_out(s):
            pltpu.make_async_copy(
                ob.at[s], out_hbm.at[:, pl.ds(0, BKJ)], sos[s]
            ).wait()

        for s in range(NSLOT):
            issue_in(s, s)

        pltpu.sync_copy(e0_hbm, e0v)
        pltpu.sync_copy(e1_hbm, e1v)
        pltpu.sync_copy(e2_hbm, e2v)

        def bi(i, carry):
            def bj(j, carry):
                def bk(k, carry):
                    v = i * (V1 * V2) + j * V2 + k
                    for c in range(D // L):
                        row = (e0v[pl.ds(i * D + c * L, L)]
                               + e1v[pl.ds(j * D + c * L, L)]
                               + e2v[pl.ds(k * D + c * L, L)])
                        plsc.store_scatter(tbl, [iota_vc + (c * L * VC + v)], row)
                    return carry
                return lax.fori_loop(0, V2, bk, 0)
            return lax.fori_loop(0, V1, bj, 0)
        lax.fori_loop(0, V0, bi, 0)

        def body(h, carry):
            for s in range(NSLOT):
                b = h * NSLOT + s

                @pl.when(b < nb)
                def _(b=b, s=s):
                    wait_in(s)

                    @pl.when(b >= NSLOT)
                    def _():
                        wait_out(s)

                    def grp(g, carry, s=s):
                        jj = g >> 3
                        c16 = (g & 7) * L
                        f = av[s, pl.ds(g * L, L)]
                        for d0 in range(0, D, 16):
                            ws = [plsc.load_gather(tbl, [f + d * VC])
                                  for d in range(d0, d0 + 16)]
                            for k, d in enumerate(range(d0, d0 + 16)):
                                ob[s, d // 8, jj, d % 8, pl.ds(c16, L)] = ws[k]
                        return carry
                    lax.fori_loop(0, BKJ * 8, grp, 0)

                    issue_out(b, s)

                    @pl.when(b + NSLOT < nb)
                    def _():
                        issue_in(b + NSLOT, s)
            return carry

        lax.fori_loop(0, (nb + NSLOT - 1) // NSLOT, body, 0)

        for s in range(NSLOT):
            @pl.when(nb > s)
            def _(s=s):
                wait_out(s)

    return _gather


@jax.jit
def kernel(edge_attr, emb0, emb1, emb2):
    E = edge_attr.shape[0]
    ea = edge_attr.astype(jnp.int32)
    f = ea[:, 0] * (V1 * V2) + ea[:, 1] * V2 + ea[:, 2]
    m = _make_gather(E)(
        emb0.reshape(V0 * D), emb1.reshape(V1 * D), emb2.reshape(V2 * D), f,
    )
    return m.transpose(1, 3, 0, 2).reshape(E, D)
